# baseline (speedup 1.0000x reference)
import functools

import jax
import jax.numpy as jnp
from jax import lax
from jax.experimental import pallas as pl
from jax.experimental.pallas import tpu as pltpu
from jax.experimental.pallas import tpu_sc as plsc

N_LEVELS = 6
N_LABELS = 2048
BATCH = 1024
ROW = 2048
ROW_W = ROW // 4

NC = 2
NS = 16
NW = NC * NS
B_PER_W = BATCH // NW
ROWS_PER_W = B_PER_W * N_LEVELS
CHUNK = 48
NCHUNK = ROWS_PER_W // CHUNK
NBUF = 2
VPC = CHUNK // 16


def _body(leaf_hbm, adv_hbm, out_hbm, leaf_v, gidx_v, oidx_v,
          rows0, rows1, gsem0, gsem1, ssem):
    wid = lax.axis_index("s") * NC + lax.axis_index("c")
    b0 = wid * B_PER_W
    pltpu.sync_copy(leaf_hbm.at[pl.ds(b0, B_PER_W)], leaf_v)

    lane = lax.broadcasted_iota(jnp.int32, (16,), 0)
    leaf_half = (leaf_v[pl.ds(0, 16)], leaf_v[pl.ds(16, 16)])
    ob_base = b0 * N_LEVELS
    rows_bufs = (rows0, rows1)
    gsems = (gsem0, gsem1)

    for v in range(ROWS_PER_W // 16):
        i = v // 2
        half = v % 2
        gvec = leaf_half[half] + i * N_LABELS
        ovec = (lane + 16 * half) * N_LEVELS + (i + ob_base)
        c, k = v // VPC, (v % VPC) * 16
        gidx_v[c, pl.ds(k, 16)] = gvec
        oidx_v[c, pl.ds(k, 16)] = ovec

    copies = [None, None]
    for c in range(NCHUNK):
        s = c % NBUF
        if copies[s] is not None:
            copies[s].wait()
            pltpu.async_copy(
                rows_bufs[s], out_hbm.at[oidx_v.at[c - NBUF]], ssem
            ).wait()
        copies[s] = pltpu.async_copy(
            adv_hbm.at[gidx_v.at[c]], rows_bufs[s], gsems[s]
        )
    for c in range(NCHUNK - NBUF, NCHUNK):
        s = c % NBUF
        copies[s].wait()
        pltpu.async_copy(rows_bufs[s], out_hbm.at[oidx_v.at[c]], ssem).wait()


@jax.jit
def _sc_gather(leaf, adv_words):
    mesh = plsc.VectorSubcoreMesh(core_axis_name="c", subcore_axis_name="s")
    f = functools.partial(
        pl.kernel,
        mesh=mesh,
        out_type=jax.ShapeDtypeStruct((BATCH * N_LEVELS, ROW_W), jnp.int32),
        scratch_types=[
            pltpu.VMEM((B_PER_W,), jnp.int32),
            pltpu.VMEM((NCHUNK, CHUNK), jnp.int32),
            pltpu.VMEM((NCHUNK, CHUNK), jnp.int32),
            pltpu.VMEM((CHUNK, ROW_W), jnp.int32),
            pltpu.VMEM((CHUNK, ROW_W), jnp.int32),
            pltpu.SemaphoreType.DMA,
            pltpu.SemaphoreType.DMA,
            pltpu.SemaphoreType.DMA,
        ],
    )(_body)
    return f(leaf, adv_words)


def kernel(labels, adversaries):
    leaf = labels[:, N_LEVELS - 1]
    adv_words = lax.bitcast_convert_type(
        adversaries.view(jnp.uint8).reshape(N_LEVELS * N_LABELS, ROW_W, 4),
        jnp.int32,
    )
    out_words = _sc_gather(leaf, adv_words)
    out_u8 = lax.bitcast_convert_type(out_words, jnp.uint8)
    return out_u8.reshape(BATCH, N_LEVELS, ROW).view(jnp.bool_)


# speedup vs baseline: 11.6882x; 11.6882x over previous
import functools

import jax
import jax.numpy as jnp
from jax import lax
from jax.experimental import pallas as pl
from jax.experimental.pallas import tpu as pltpu
from jax.experimental.pallas import tpu_sc as plsc

N_LEVELS = 6
N_LABELS = 2048
BATCH = 1024
ROW = 2048

NC = 2
NS = 16
NW = NC * NS
B_PER_W = BATCH // NW
ROWS_PER_W = B_PER_W * N_LEVELS
CHUNK = 16
NCHUNK = ROWS_PER_W // CHUNK
NBUF = 3


def _body(leaf_hbm, adv_hbm, out_hbm, leaf_v, gidx_v, oidx_v,
          rows0, rows1, rows2, gsem0, gsem1, gsem2, ssem):
    wid = lax.axis_index("s") * NC + lax.axis_index("c")
    b0 = wid * B_PER_W
    pltpu.sync_copy(leaf_hbm.at[pl.ds(b0, B_PER_W)], leaf_v)

    lane = lax.broadcasted_iota(jnp.int32, (16,), 0)
    leaf_half = (leaf_v[pl.ds(0, 16)], leaf_v[pl.ds(16, 16)])
    ob_base = b0 * N_LEVELS
    rows_bufs = (rows0, rows1, rows2)
    gsems = (gsem0, gsem1, gsem2)

    for v in range(ROWS_PER_W // 16):
        i = v // 2
        half = v % 2
        gidx_v[v, pl.ds(0, 16)] = leaf_half[half] + i * N_LABELS
        oidx_v[v, pl.ds(0, 16)] = (lane + 16 * half) * N_LEVELS + (i + ob_base)

    copies = [None] * NBUF
    for c in range(NCHUNK):
        s = c % NBUF
        if copies[s] is not None:
            copies[s].wait()
            pltpu.async_copy(
                rows_bufs[s], out_hbm.at[oidx_v.at[c - NBUF]], ssem
            ).wait()
        copies[s] = pltpu.async_copy(
            adv_hbm.at[gidx_v.at[c]], rows_bufs[s], gsems[s]
        )
    for c in range(NCHUNK - NBUF, NCHUNK):
        s = c % NBUF
        copies[s].wait()
        pltpu.async_copy(rows_bufs[s], out_hbm.at[oidx_v.at[c]], ssem).wait()


@jax.jit
def _sc_gather(leaf, adv2d):
    mesh = plsc.VectorSubcoreMesh(core_axis_name="c", subcore_axis_name="s")
    f = functools.partial(
        pl.kernel,
        mesh=mesh,
        out_type=jax.ShapeDtypeStruct((BATCH * N_LEVELS, ROW), jnp.bool_),
        scratch_types=[
            pltpu.VMEM((B_PER_W,), jnp.int32),
            pltpu.VMEM((NCHUNK, CHUNK), jnp.int32),
            pltpu.VMEM((NCHUNK, CHUNK), jnp.int32),
            pltpu.VMEM((CHUNK, ROW), jnp.bool_),
            pltpu.VMEM((CHUNK, ROW), jnp.bool_),
            pltpu.VMEM((CHUNK, ROW), jnp.bool_),
            pltpu.SemaphoreType.DMA,
            pltpu.SemaphoreType.DMA,
            pltpu.SemaphoreType.DMA,
            pltpu.SemaphoreType.DMA,
        ],
    )(_body)
    return f(leaf, adv2d)


def kernel(labels, adversaries):
    leaf = labels[:, N_LEVELS - 1]
    adv2d = adversaries.reshape(N_LEVELS * N_LABELS, ROW)
    out2d = _sc_gather(leaf, adv2d)
    return out2d.reshape(BATCH, N_LEVELS, ROW)


# speedup vs baseline: 18.0850x; 1.5473x over previous
import functools

import jax
import jax.numpy as jnp
from jax import lax
from jax.experimental import pallas as pl
from jax.experimental.pallas import tpu as pltpu
from jax.experimental.pallas import tpu_sc as plsc

N_LEVELS = 6
N_LABELS = 2048
BATCH = 1024
ROW = 2048

NC = 2
NS = 16
NW = NC * NS
B_PER_W = BATCH // NW
ROWS_PER_W = B_PER_W * N_LEVELS
CHUNK = 16
NCHUNK = ROWS_PER_W // CHUNK
NBUF = 3


def _body(leaf_hbm, adv_hbm, out_hbm, leaf_v, gidx_v,
          rows0, rows1, rows2, gsem0, gsem1, gsem2):
    wid = lax.axis_index("s") * NC + lax.axis_index("c")
    b0 = wid * B_PER_W
    pltpu.sync_copy(leaf_hbm.at[pl.ds(b0, B_PER_W)], leaf_v)

    leaf_half = (leaf_v[pl.ds(0, 16)], leaf_v[pl.ds(16, 16)])
    rows_bufs = (rows0, rows1, rows2)
    gsems = (gsem0, gsem1, gsem2)

    for c in range(NCHUNK):
        gidx_v[c, pl.ds(0, 16)] = leaf_half[c % 2] + (c // 2) * N_LABELS

    copies = [None] * NBUF
    for c in range(NCHUNK + NBUF):
        s = c % NBUF
        if c >= NBUF:
            d = c - NBUF
            copies[s].wait()
            out_base = (d // 2) * BATCH + b0 + (d % 2) * CHUNK
            pltpu.sync_copy(rows_bufs[s], out_hbm.at[pl.ds(out_base, CHUNK)])
        if c < NCHUNK:
            copies[s] = pltpu.async_copy(
                adv_hbm.at[gidx_v.at[c]], rows_bufs[s], gsems[s]
            )


@jax.jit
def _sc_gather(leaf, adv_words):
    mesh = plsc.VectorSubcoreMesh(core_axis_name="c", subcore_axis_name="s")
    f = functools.partial(
        pl.kernel,
        mesh=mesh,
        out_type=jax.ShapeDtypeStruct((N_LEVELS * BATCH, ROW), jnp.int32),
        scratch_types=[
            pltpu.VMEM((B_PER_W,), jnp.int32),
            pltpu.VMEM((NCHUNK, CHUNK), jnp.int32),
            pltpu.VMEM((CHUNK, ROW), jnp.int32),
            pltpu.VMEM((CHUNK, ROW), jnp.int32),
            pltpu.VMEM((CHUNK, ROW), jnp.int32),
            pltpu.SemaphoreType.DMA,
            pltpu.SemaphoreType.DMA,
            pltpu.SemaphoreType.DMA,
        ],
    )(_body)
    return f(leaf, adv_words)


def kernel(labels, adversaries):
    leaf = labels[:, N_LEVELS - 1]
    adv_words = adversaries.astype(jnp.int32).reshape(N_LEVELS * N_LABELS, ROW)
    out_words = _sc_gather(leaf, adv_words)
    out = out_words.reshape(N_LEVELS, BATCH, ROW) != 0
    return out.transpose(1, 0, 2)
